# baseline (speedup 1.0000x reference)
import functools

import jax
import jax.numpy as jnp
from jax import lax
from jax.experimental import pallas as pl
from jax.experimental.pallas import tpu as pltpu
from jax.experimental.pallas import tpu_sc as plsc

NUM_TRICKS = 100000
NUM_DRAFT_TRICKS = 1000
NUM_EMBEDDINGS = NUM_TRICKS + NUM_DRAFT_TRICKS
DRAFT_DELTA = NUM_TRICKS
DRAFT_PHASE = 2
EMBED_DIM = 128

NUM_WORKERS = 32
LANES = 16
CHUNK = 128


@functools.lru_cache(maxsize=None)
def _build(n_total):
    n = n_total // NUM_WORKERS
    n_chunks = n // CHUNK
    mesh = plsc.VectorSubcoreMesh(core_axis_name="c", subcore_axis_name="s")

    @functools.partial(
        pl.kernel,
        mesh=mesh,
        out_type=jax.ShapeDtypeStruct((n_total, EMBED_DIM), jnp.float32),
        scratch_types=[
            pltpu.VMEM((n,), jnp.int32),
            pltpu.VMEM((n,), jnp.int32),
            pltpu.VMEM((n,), jnp.int32),
            pltpu.VMEM((CHUNK, EMBED_DIM), jnp.float32),
            pltpu.SemaphoreType.DMA,
        ],
    )
    def kern(trick_hbm, phase_hbm, table_hbm, out_hbm,
             trick_v, phase_v, idx_v, rows_v, sem):
        wid = lax.axis_index("s") * 2 + lax.axis_index("c")
        base = wid * n
        pltpu.sync_copy(trick_hbm.at[pl.ds(base, n)], trick_v)
        pltpu.sync_copy(phase_hbm.at[pl.ds(base, n)], phase_v)

        def idx_body(i, carry):
            o = i * LANES
            t = trick_v[pl.ds(o, LANES)]
            p = phase_v[pl.ds(o, LANES)]
            t = t + jnp.where(p == DRAFT_PHASE, DRAFT_DELTA, 0)
            t = jnp.clip(t, -1, NUM_EMBEDDINGS - 1) + 1
            idx_v[pl.ds(o, LANES)] = t
            return carry

        lax.fori_loop(0, n // LANES, idx_body, 0)

        def chunk_body(c, carry):
            o = c * CHUNK
            pltpu.async_copy(
                table_hbm.at[idx_v.at[pl.ds(o, CHUNK)]], rows_v, sem
            ).wait()
            pltpu.sync_copy(rows_v, out_hbm.at[pl.ds(base + o, CHUNK)])
            return carry

        lax.fori_loop(0, n_chunks, chunk_body, 0)

    return kern


def kernel(trick, phase, table):
    b, h = trick.shape
    n_total = b * h
    out = _build(n_total)(
        trick.reshape(n_total).astype(jnp.int32),
        phase.reshape(n_total).astype(jnp.int32),
        table,
    )
    return out.reshape(b, h, EMBED_DIM)
